# baseline (speedup 1.0000x reference)
import functools

import jax
import jax.numpy as jnp
from jax import lax
from jax.experimental import pallas as pl
from jax.experimental.pallas import tpu as pltpu
from jax.experimental.pallas import tpu_sc as plsc

_NC, _NS = 2, 16
_NW = _NC * _NS
_NBUF = 2
_CHUNK = 4


def _sc_gather_body(uniq_hbm, rl_hbm, pos_hbm, tgt_hbm, nch_hbm, table_hbm,
                    out_hbm, tv_hbm, uniq_v, rl_v, pos_v, tgt_v, nch_v,
                    tvacc_vm, bufs, gsems, ssems, *, cap, C):
    wid = lax.axis_index("s") * _NC + lax.axis_index("c")
    pltpu.sync_copy(uniq_hbm.at[wid], uniq_v)
    pltpu.sync_copy(rl_hbm.at[wid], rl_v)
    pltpu.sync_copy(pos_hbm.at[wid], pos_v)
    pltpu.sync_copy(tgt_hbm.at[wid], tgt_v)
    pltpu.sync_copy(nch_hbm.at[wid], nch_v)
    n_chunks = nch_v[pl.ds(0, 16)][0]
    lane16 = lax.iota(jnp.int32, 16)
    lane0 = lane16 == 0

    for b in range(_NBUF):
        pltpu.async_copy(table_hbm.at[uniq_v.at[b]], bufs[b], gsems[b])

    def _step(i, b, carry):
        pcur, acc = carry
        pltpu.make_async_copy(
            table_hbm.at[uniq_v.at[i]], bufs[b], gsems[b]).wait()
        pc0 = pcur
        for j in range(_CHUNK):
            rl = rl_v[pl.ds(i * _CHUNK + j, 16)][0]

            def _run(t, c):
                pc, a = c
                p = pos_v[pl.ds(pc, 16)][0]
                pltpu.async_copy(bufs[b].at[pl.ds(j, 1)],
                                 out_hbm.at[pl.ds(p, 1)], ssems[b])
                tg = tgt_v[pl.ds(pc, 16)][0]
                vals = plsc.load_gather(
                    bufs[b], [jnp.full((16,), j, jnp.int32),
                              jnp.full((16,), tg, jnp.int32)])
                a = a + jnp.where(lane0, vals, 0.0)
                return (pc + 1, a)

            pcur, acc = lax.fori_loop(0, rl, _run, (pcur, acc))

        def _drain(t, c):
            pltpu.make_async_copy(bufs[b], out_hbm.at[pl.ds(0, _CHUNK)],
                                  ssems[b]).wait()
            return c

        lax.fori_loop(0, (pcur - pc0) // _CHUNK, _drain, 0)

        def _drain1(t, c):
            pltpu.make_async_copy(bufs[b].at[pl.ds(0, 1)],
                                  out_hbm.at[pl.ds(0, 1)], ssems[b]).wait()
            return c

        lax.fori_loop(0, (pcur - pc0) % _CHUNK, _drain1, 0)
        nxt = i + _NBUF

        @pl.when(nxt < n_chunks)
        def _():
            pltpu.async_copy(table_hbm.at[uniq_v.at[nxt]], bufs[b], gsems[b])
        return (pcur, acc)

    def _loop_body(g, carry):
        for b in range(_NBUF):
            carry = _step(g + b, b, carry)
        return carry

    init = (jnp.int32(0), jnp.zeros((16,), jnp.float32))
    _, final_acc = pl.loop(0, n_chunks, step=_NBUF,
                           init_carry=init)(_loop_body)
    tvacc_vm[...] = final_acc
    pltpu.sync_copy(tvacc_vm, tv_hbm.at[wid])


def _sc_gather(uniq3, rl2, pos2, tgt2, nch2, table):
    cap = rl2.shape[1] - 16
    C = table.shape[1]
    n_rows = _NW * cap
    mesh = plsc.VectorSubcoreMesh(
        core_axis_name="c", subcore_axis_name="s",
        num_cores=_NC, num_subcores=_NS)
    body = functools.partial(_sc_gather_body, cap=cap, C=C)

    def wrapped(uniq_hbm, rl_hbm, pos_hbm, tgt_hbm, nch_hbm, table_hbm,
                out_hbm, tv_hbm, *scratch):
        bufs = scratch[:_NBUF]
        gsems = scratch[_NBUF:2 * _NBUF]
        ssems = scratch[2 * _NBUF:3 * _NBUF]
        body(uniq_hbm, rl_hbm, pos_hbm, tgt_hbm, nch_hbm, table_hbm,
             out_hbm, tv_hbm, *scratch[3 * _NBUF:3 * _NBUF + 6],
             bufs, gsems, ssems)

    return pl.kernel(
        wrapped,
        out_type=(jax.ShapeDtypeStruct((n_rows, C), jnp.float32),
                  jax.ShapeDtypeStruct((_NW, 16), jnp.float32)),
        mesh=mesh,
        compiler_params=pltpu.CompilerParams(needs_layout_passes=False),
        scratch_types=(
            [pltpu.VMEM((_CHUNK, C), jnp.float32)] * _NBUF
            + [pltpu.SemaphoreType.DMA] * (2 * _NBUF)
            + [pltpu.VMEM((cap // _CHUNK, _CHUNK), jnp.int32),
               pltpu.VMEM((cap + 16,), jnp.int32),
               pltpu.VMEM((cap + 16,), jnp.int32),
               pltpu.VMEM((cap + 16,), jnp.int32),
               pltpu.VMEM((16,), jnp.int32),
               pltpu.VMEM((16,), jnp.float32)]
        ),
    )(uniq3, rl2, pos2, tgt2, nch2, table)


def _lse_body(rows_ref, lse_ref):
    rows = rows_ref[...]
    m = jnp.max(rows, axis=1, keepdims=True)
    e = jnp.exp(rows - m)
    s = jnp.sum(e, axis=1, keepdims=True)
    lse_ref[...] = m + jnp.log(s)


def _tc_lse(table):
    V, C = table.shape
    R = 256
    lse = pl.pallas_call(
        _lse_body,
        grid=(V // R,),
        in_specs=[pl.BlockSpec((R, C), lambda i: (i, 0))],
        out_specs=pl.BlockSpec((R, 1), lambda i: (i, 0)),
        out_shape=jax.ShapeDtypeStruct((V, 1), jnp.float32),
    )(table)
    return lse.reshape(V)


def _nll_body(idxe_hbm, lse_hbm, out_hbm, idxe_v, lse_buf, acc_vm, sem,
              *, rows_per_w):
    wid = lax.axis_index("s") * _NC + lax.axis_index("c")
    n_dma = rows_per_w // 128
    pltpu.sync_copy(idxe_hbm.at[wid], idxe_v)
    for j in range(n_dma):
        pltpu.async_copy(lse_hbm.at[idxe_v.at[j]],
                         lse_buf.at[pl.ds(j * 128, 128)], sem)
    for j in range(n_dma):
        pltpu.make_async_copy(lse_hbm.at[idxe_v.at[j]],
                              lse_buf.at[pl.ds(j * 128, 128)], sem).wait()
    acc = jnp.zeros((16,), jnp.float32)
    for k in range(rows_per_w // 16):
        acc = acc + lse_buf[pl.ds(k * 16, 16)]
    acc_vm[...] = acc
    pltpu.sync_copy(acc_vm, out_hbm.at[wid])


def _sc_lse_partials(idx_flat, lse):
    n_rows = idx_flat.shape[0]
    rows_per_w = n_rows // _NW
    n_dma = rows_per_w // 128
    idx3 = idx_flat.reshape(_NW, n_dma, 128)
    mesh = plsc.VectorSubcoreMesh(
        core_axis_name="c", subcore_axis_name="s",
        num_cores=_NC, num_subcores=_NS)
    return pl.kernel(
        functools.partial(_nll_body, rows_per_w=rows_per_w),
        out_type=jax.ShapeDtypeStruct((_NW, 16), jnp.float32),
        mesh=mesh,
        scratch_types=(
            pltpu.VMEM((n_dma, 128), jnp.int32),
            pltpu.VMEM((rows_per_w,), jnp.float32),
            pltpu.VMEM((16,), jnp.float32),
            pltpu.SemaphoreType.DMA,
        ),
    )(idx3, lse)


def _mean_body(lse_part_ref, tv_part_ref, loss_ref, *, n_rows):
    nll_sum = jnp.sum(lse_part_ref[...]) - jnp.sum(tv_part_ref[...])
    loss_ref[...] = jnp.full((1, 1), nll_sum / jnp.float32(n_rows),
                             jnp.float32)


def _plan(idx_flat, tgt_flat):
    N = idx_flat.shape[0]
    cap = N // _NW
    pos0 = jnp.arange(N, dtype=jnp.int32)
    s, order, tgt_sorted = lax.sort((idx_flat, pos0, tgt_flat), num_keys=1)
    s2 = s.reshape(_NW, cap)
    first = jnp.concatenate(
        [jnp.ones((_NW, 1), jnp.bool_), s2[:, 1:] != s2[:, :-1]], axis=1)
    slot = jnp.cumsum(first, axis=1).astype(jnp.int32) - 1
    n_uniq = slot[:, -1] + 1
    wpos = jnp.broadcast_to(jnp.arange(cap, dtype=jnp.int32)[None, :],
                            (_NW, cap))
    key = jnp.where(first, slot, cap)
    _, uniq, bpos = lax.sort((key, s2, wpos), dimension=-1, num_keys=1)
    nxt = jnp.concatenate(
        [bpos[:, 1:], jnp.full((_NW, 1), cap, jnp.int32)], axis=1)
    nextb = jnp.where(wpos == n_uniq[:, None] - 1, cap, nxt)
    rl = jnp.where(wpos < n_uniq[:, None], nextb - bpos, 0)
    nch = (n_uniq + _CHUNK - 1) // _CHUNK
    nch = ((nch + _NBUF - 1) // _NBUF) * _NBUF
    nch = jnp.minimum(nch, cap // _CHUNK)
    nch16 = jnp.broadcast_to(nch[:, None], (_NW, 16))
    pad16 = ((0, 0), (0, 16))
    rl_p = jnp.pad(rl, pad16)
    pos2 = jnp.pad(order.reshape(_NW, cap), pad16)
    tgt2 = jnp.pad(tgt_sorted.reshape(_NW, cap), pad16)
    return (uniq.reshape(_NW, cap // _CHUNK, _CHUNK), rl_p, pos2, tgt2,
            nch16)


def kernel(idx, targets, table):
    B, T = idx.shape
    N = B * T
    idx_flat = idx.reshape(N).astype(jnp.int32)
    tgt_flat = targets.reshape(N).astype(jnp.int32)

    uniq3, rl2, pos2, tgt2, nch8 = _plan(idx_flat, tgt_flat)
    logits, tv_partials = _sc_gather(uniq3, rl2, pos2, tgt2, nch8, table)
    lse = _tc_lse(table)
    lse_partials = _sc_lse_partials(idx_flat, lse)
    loss = pl.pallas_call(
        functools.partial(_mean_body, n_rows=N),
        out_shape=jax.ShapeDtypeStruct((1, 1), jnp.float32),
    )(lse_partials, tv_partials)
    return (logits, loss[0, 0])


# speedup vs baseline: 1.0251x; 1.0251x over previous
import functools

import jax
import jax.numpy as jnp
from jax import lax
from jax.experimental import pallas as pl
from jax.experimental.pallas import tpu as pltpu
from jax.experimental.pallas import tpu_sc as plsc

_NC, _NS = 2, 16
_NW = _NC * _NS
_NBUF = 2
_CHUNK = 2


def _sc_gather_body(uniq_hbm, rl_hbm, pos_hbm, tgt_hbm, nch_hbm, table_hbm,
                    out_hbm, tv_hbm, uniq_v, rl_v, pos_v, tgt_v, nch_v,
                    tvacc_vm, bufs, gsems, ssems, *, cap, C):
    wid = lax.axis_index("s") * _NC + lax.axis_index("c")
    pltpu.sync_copy(uniq_hbm.at[wid], uniq_v)
    pltpu.sync_copy(rl_hbm.at[wid], rl_v)
    pltpu.sync_copy(pos_hbm.at[wid], pos_v)
    pltpu.sync_copy(tgt_hbm.at[wid], tgt_v)
    pltpu.sync_copy(nch_hbm.at[wid], nch_v)
    n_chunks = nch_v[pl.ds(0, 16)][0]
    lane16 = lax.iota(jnp.int32, 16)
    lane0 = lane16 == 0

    for b in range(_NBUF):
        pltpu.async_copy(table_hbm.at[uniq_v.at[b]], bufs[b], gsems[b])

    def _step(i, b, carry):
        pcur, acc = carry
        pltpu.make_async_copy(
            table_hbm.at[uniq_v.at[i]], bufs[b], gsems[b]).wait()
        pc0 = pcur
        for j in range(_CHUNK):
            rl = rl_v[pl.ds(i * _CHUNK + j, 16)][0]

            def _run(t, c):
                pc, a = c
                p = pos_v[pl.ds(pc, 16)][0]
                pltpu.async_copy(bufs[b].at[pl.ds(j, 1)],
                                 out_hbm.at[pl.ds(p, 1)], ssems[b])
                tg = tgt_v[pl.ds(pc, 16)][0]
                vals = plsc.load_gather(
                    bufs[b], [jnp.full((16,), j, jnp.int32),
                              jnp.full((16,), tg, jnp.int32)])
                a = a + jnp.where(lane0, vals, 0.0)
                return (pc + 1, a)

            pcur, acc = lax.fori_loop(0, rl, _run, (pcur, acc))

        def _drain(t, c):
            pltpu.make_async_copy(bufs[b], out_hbm.at[pl.ds(0, _CHUNK)],
                                  ssems[b]).wait()
            return c

        lax.fori_loop(0, (pcur - pc0) // _CHUNK, _drain, 0)

        def _drain1(t, c):
            pltpu.make_async_copy(bufs[b].at[pl.ds(0, 1)],
                                  out_hbm.at[pl.ds(0, 1)], ssems[b]).wait()
            return c

        lax.fori_loop(0, (pcur - pc0) % _CHUNK, _drain1, 0)
        nxt = i + _NBUF

        @pl.when(nxt < n_chunks)
        def _():
            pltpu.async_copy(table_hbm.at[uniq_v.at[nxt]], bufs[b], gsems[b])
        return (pcur, acc)

    def _loop_body(g, carry):
        for b in range(_NBUF):
            carry = _step(g + b, b, carry)
        return carry

    init = (jnp.int32(0), jnp.zeros((16,), jnp.float32))
    _, final_acc = pl.loop(0, n_chunks, step=_NBUF,
                           init_carry=init)(_loop_body)
    tvacc_vm[...] = final_acc
    pltpu.sync_copy(tvacc_vm, tv_hbm.at[wid])


def _sc_gather(uniq3, rl2, pos2, tgt2, nch2, table):
    cap = rl2.shape[1] - 16
    C = table.shape[1]
    n_rows = _NW * cap
    mesh = plsc.VectorSubcoreMesh(
        core_axis_name="c", subcore_axis_name="s",
        num_cores=_NC, num_subcores=_NS)
    body = functools.partial(_sc_gather_body, cap=cap, C=C)

    def wrapped(uniq_hbm, rl_hbm, pos_hbm, tgt_hbm, nch_hbm, table_hbm,
                out_hbm, tv_hbm, *scratch):
        bufs = scratch[:_NBUF]
        gsems = scratch[_NBUF:2 * _NBUF]
        ssems = scratch[2 * _NBUF:3 * _NBUF]
        body(uniq_hbm, rl_hbm, pos_hbm, tgt_hbm, nch_hbm, table_hbm,
             out_hbm, tv_hbm, *scratch[3 * _NBUF:3 * _NBUF + 6],
             bufs, gsems, ssems)

    return pl.kernel(
        wrapped,
        out_type=(jax.ShapeDtypeStruct((n_rows, C), jnp.float32),
                  jax.ShapeDtypeStruct((_NW, 16), jnp.float32)),
        mesh=mesh,
        compiler_params=pltpu.CompilerParams(needs_layout_passes=False),
        scratch_types=(
            [pltpu.VMEM((_CHUNK, C), jnp.float32)] * _NBUF
            + [pltpu.SemaphoreType.DMA] * (2 * _NBUF)
            + [pltpu.VMEM((cap // _CHUNK, _CHUNK), jnp.int32),
               pltpu.VMEM((cap + 16,), jnp.int32),
               pltpu.VMEM((cap + 16,), jnp.int32),
               pltpu.VMEM((cap + 16,), jnp.int32),
               pltpu.VMEM((16,), jnp.int32),
               pltpu.VMEM((16,), jnp.float32)]
        ),
    )(uniq3, rl2, pos2, tgt2, nch2, table)


def _lse_body(dep_ref, rows_ref, lse_ref):
    del dep_ref
    rows = rows_ref[...]
    m = jnp.max(rows, axis=1, keepdims=True)
    e = jnp.exp(rows - m)
    s = jnp.sum(e, axis=1, keepdims=True)
    lse_ref[...] = m + jnp.log(s)


def _tc_lse(table, dep):
    V, C = table.shape
    R = 256
    lse = pl.pallas_call(
        _lse_body,
        grid=(V // R,),
        in_specs=[pl.BlockSpec((_NW, 16), lambda i: (0, 0)),
                  pl.BlockSpec((R, C), lambda i: (i, 0))],
        out_specs=pl.BlockSpec((R, 1), lambda i: (i, 0)),
        out_shape=jax.ShapeDtypeStruct((V, 1), jnp.float32),
    )(dep, table)
    return lse.reshape(V)


def _nll_body(idxe_hbm, lse_hbm, out_hbm, idxe_v, lse_buf, acc_vm, sem,
              *, rows_per_w):
    wid = lax.axis_index("s") * _NC + lax.axis_index("c")
    n_dma = rows_per_w // 128
    pltpu.sync_copy(idxe_hbm.at[wid], idxe_v)
    for j in range(n_dma):
        pltpu.async_copy(lse_hbm.at[idxe_v.at[j]],
                         lse_buf.at[pl.ds(j * 128, 128)], sem)
    for j in range(n_dma):
        pltpu.make_async_copy(lse_hbm.at[idxe_v.at[j]],
                              lse_buf.at[pl.ds(j * 128, 128)], sem).wait()
    acc = jnp.zeros((16,), jnp.float32)
    for k in range(rows_per_w // 16):
        acc = acc + lse_buf[pl.ds(k * 16, 16)]
    acc_vm[...] = acc
    pltpu.sync_copy(acc_vm, out_hbm.at[wid])


def _sc_lse_partials(idx_flat, lse):
    n_rows = idx_flat.shape[0]
    rows_per_w = n_rows // _NW
    n_dma = rows_per_w // 128
    idx3 = idx_flat.reshape(_NW, n_dma, 128)
    mesh = plsc.VectorSubcoreMesh(
        core_axis_name="c", subcore_axis_name="s",
        num_cores=_NC, num_subcores=_NS)
    return pl.kernel(
        functools.partial(_nll_body, rows_per_w=rows_per_w),
        out_type=jax.ShapeDtypeStruct((_NW, 16), jnp.float32),
        mesh=mesh,
        scratch_types=(
            pltpu.VMEM((n_dma, 128), jnp.int32),
            pltpu.VMEM((rows_per_w,), jnp.float32),
            pltpu.VMEM((16,), jnp.float32),
            pltpu.SemaphoreType.DMA,
        ),
    )(idx3, lse)


def _mean_body(lse_part_ref, tv_part_ref, loss_ref, *, n_rows):
    nll_sum = jnp.sum(lse_part_ref[...]) - jnp.sum(tv_part_ref[...])
    loss_ref[...] = jnp.full((1, 1), nll_sum / jnp.float32(n_rows),
                             jnp.float32)


def _plan(idx_flat, tgt_flat):
    N = idx_flat.shape[0]
    cap = N // _NW
    pos0 = jnp.arange(N, dtype=jnp.int32)
    s, order, tgt_sorted = lax.sort((idx_flat, pos0, tgt_flat), num_keys=1)
    s2 = s.reshape(_NW, cap)
    first = jnp.concatenate(
        [jnp.ones((_NW, 1), jnp.bool_), s2[:, 1:] != s2[:, :-1]], axis=1)
    slot = jnp.cumsum(first, axis=1).astype(jnp.int32) - 1
    n_uniq = slot[:, -1] + 1
    wpos = jnp.broadcast_to(jnp.arange(cap, dtype=jnp.int32)[None, :],
                            (_NW, cap))
    key = jnp.where(first, slot, cap)
    _, uniq, bpos = lax.sort((key, s2, wpos), dimension=-1, num_keys=1)
    nxt = jnp.concatenate(
        [bpos[:, 1:], jnp.full((_NW, 1), cap, jnp.int32)], axis=1)
    nextb = jnp.where(wpos == n_uniq[:, None] - 1, cap, nxt)
    rl = jnp.where(wpos < n_uniq[:, None], nextb - bpos, 0)
    nch = (n_uniq + _CHUNK - 1) // _CHUNK
    nch = ((nch + _NBUF - 1) // _NBUF) * _NBUF
    nch = jnp.minimum(nch, cap // _CHUNK)
    nch16 = jnp.broadcast_to(nch[:, None], (_NW, 16))
    pad16 = ((0, 0), (0, 16))
    rl_p = jnp.pad(rl, pad16)
    pos2 = jnp.pad(order.reshape(_NW, cap), pad16)
    tgt2 = jnp.pad(tgt_sorted.reshape(_NW, cap), pad16)
    return (uniq.reshape(_NW, cap // _CHUNK, _CHUNK), rl_p, pos2, tgt2,
            nch16)


def kernel(idx, targets, table):
    B, T = idx.shape
    N = B * T
    idx_flat = idx.reshape(N).astype(jnp.int32)
    tgt_flat = targets.reshape(N).astype(jnp.int32)

    uniq3, rl2, pos2, tgt2, nch8 = _plan(idx_flat, tgt_flat)
    logits, tv_partials = _sc_gather(uniq3, rl2, pos2, tgt2, nch8, table)
    lse = _tc_lse(table, nch8)
    lse_partials = _sc_lse_partials(idx_flat, lse)
    loss = pl.pallas_call(
        functools.partial(_mean_body, n_rows=N),
        out_shape=jax.ShapeDtypeStruct((1, 1), jnp.float32),
    )(lse_partials, tv_partials)
    return (logits, loss[0, 0])
